# baseline (speedup 1.0000x reference)
import functools

import jax
import jax.numpy as jnp
from jax import lax
from jax.experimental import pallas as pl
from jax.experimental.pallas import tpu as pltpu
from jax.experimental.pallas import tpu_sc as plsc

_B = 16384
_L = 50
_D = 64
_V = 1000000
_VP = _V // 2

_NC = 2
_NS = 16
_NW = _NC * _NS
_BPW = _B // _NW
_CB = 4
_CHUNK = _CB * _L
_NCHUNK = _BPW // _CB
_GSUB = 40
_TP = 17
_IDXPAD = 224
_CPAD = _BPW + 16


def _make_sc_kernel():
    mesh = plsc.VectorSubcoreMesh(core_axis_name="c", subcore_axis_name="s")

    @functools.partial(
        pl.kernel,
        mesh=mesh,
        compiler_params=pltpu.CompilerParams(needs_layout_passes=False,
                                             use_tc_tiling_on_sc=True),
        out_type=jax.ShapeDtypeStruct((_B * _L,), jnp.float32),
        scratch_types=[
            pltpu.VMEM((_CPAD,), jnp.int32),
            pltpu.VMEM((_CPAD,), jnp.int32),
            pltpu.VMEM((_BPW, 128), jnp.float32),
            pltpu.VMEM((_IDXPAD,), jnp.int32),
            pltpu.VMEM((_IDXPAD,), jnp.int32),
            pltpu.VMEM((_CHUNK, 128), jnp.float32),
            pltpu.VMEM((16 * _TP,), jnp.float32),
            pltpu.VMEM((_CHUNK,), jnp.float32),
            pltpu.SemaphoreType.DMA,
        ],
    )
    def sc_kernel(center_hbm, ctx_hbm, cen_tab, ctx_tab, out_hbm,
                  idx_c, idx_c2, v_rows, idx_v, idx_v2, u_rows,
                  tbuf, res, sem):
        wid = lax.axis_index("s") * _NC + lax.axis_index("c")
        iota = lax.iota(jnp.int32, 16)

        pltpu.sync_copy(center_hbm.at[pl.ds(wid * _BPW, _BPW)],
                        idx_c.at[pl.ds(0, _BPW)])
        for k in range(_BPW // 16):
            idx_c2[pl.ds(k * 16, 16)] = lax.shift_right_logical(
                idx_c[pl.ds(k * 16, 16)], 1)
        cps = [
            pltpu.async_copy(cen_tab.at[idx_c2.at[pl.ds(k * 128, 128)]],
                             v_rows.at[pl.ds(k * 128, 128), :], sem)
            for k in range(_BPW // 128)
        ]
        for cp in cps:
            cp.wait()

        base_w = wid * (_BPW * _L)

        def chunk_body(g, carry):
            base = base_w + g * _CHUNK
            pltpu.sync_copy(ctx_hbm.at[pl.ds(base, _CHUNK)],
                            idx_v.at[pl.ds(0, _CHUNK)])
            for k in range(_IDXPAD // 16):
                idx_v2[pl.ds(k * 16, 16)] = lax.shift_right_logical(
                    idx_v[pl.ds(k * 16, 16)], 1)
            gs = [
                pltpu.async_copy(ctx_tab.at[idx_v2.at[pl.ds(j * _GSUB, _GSUB)]],
                                 u_rows.at[pl.ds(j * _GSUB, _GSUB), :], sem)
                for j in range(_CHUNK // _GSUB)
            ]
            for cp in gs:
                cp.wait()

            gb0 = g * _CB
            pcvec = (idx_c[pl.ds(gb0, 16)] & 1) * 64
            for b in range(_CB):
                gb = gb0 + b
                pc = pcvec[b]
                vv = [v_rows[gb, pl.ds(pc + dc * 16, 16)] for dc in range(4)]
                for l0, nl in ((0, 16), (16, 16), (32, 16), (48, 2)):
                    puvec = (idx_v[pl.ds(b * _L + l0, 16)] & 1) * 64
                    for li in range(nl):
                        row = b * _L + l0 + li
                        pu = puvec[li]
                        p = u_rows[row, pl.ds(pu, 16)] * vv[0]
                        for dc in range(1, 4):
                            p = p + u_rows[row, pl.ds(pu + dc * 16, 16)] * vv[dc]
                        plsc.store_scatter(tbuf, [iota + li * _TP], p)
                    acc = plsc.load_gather(tbuf, [iota * _TP])
                    for jj in range(1, 16):
                        acc = acc + plsc.load_gather(tbuf, [iota * _TP + jj])
                    oidx = iota + (b * _L + l0)
                    if nl == 16:
                        plsc.store_scatter(res, [oidx], acc)
                    else:
                        m = iota < nl
                        plsc.store_scatter(res, [jnp.where(m, oidx, 0)], acc,
                                           mask=m)
            pltpu.sync_copy(res, out_hbm.at[pl.ds(base, _CHUNK)])
            return carry

        lax.fori_loop(0, _NCHUNK, chunk_body, 0)

    return sc_kernel


@functools.cache
def _sc_kernel_cached():
    return _make_sc_kernel()


def kernel(center, contexts_and_negatives, central_table, context_table):
    cen_p = central_table[:_V].reshape(_VP, 128)
    ctx_p = context_table[:_V].reshape(_VP, 128)
    out_flat = _sc_kernel_cached()(center.reshape(-1),
                                   contexts_and_negatives.reshape(-1),
                                   cen_p, ctx_p)
    return out_flat.reshape(_B, 1, _L)


# speedup vs baseline: 1.1387x; 1.1387x over previous
import functools

import jax
import jax.numpy as jnp
from jax import lax
from jax.experimental import pallas as pl
from jax.experimental.pallas import tpu as pltpu
from jax.experimental.pallas import tpu_sc as plsc

_B = 16384
_L = 50
_D = 64
_V = 1000000
_VPAD = 1000008
_W = 128

_NC = 2
_NS = 16
_NW = _NC * _NS
_BPW = _B // _NW
_CB = 4
_CHUNK = _CB * _L
_NCHUNK = _BPW // _CB
_GSUB = 40
_TP = 17


def _make_sc_kernel():
    mesh = plsc.VectorSubcoreMesh(core_axis_name="c", subcore_axis_name="s")

    @functools.partial(
        pl.kernel,
        mesh=mesh,
        compiler_params=pltpu.CompilerParams(needs_layout_passes=False,
                                             use_tc_tiling_on_sc=True),
        out_type=jax.ShapeDtypeStruct((_B * _L,), jnp.float32),
        scratch_types=[
            pltpu.VMEM((_BPW,), jnp.int32),
            pltpu.VMEM((_BPW, _W), jnp.float32),
            pltpu.VMEM((_CHUNK,), jnp.int32),
            pltpu.VMEM((_CHUNK,), jnp.int32),
            pltpu.VMEM((_CHUNK, _W), jnp.float32),
            pltpu.VMEM((_CHUNK, _W), jnp.float32),
            pltpu.VMEM((16 * _TP,), jnp.float32),
            pltpu.VMEM((_CHUNK,), jnp.float32),
            pltpu.VMEM((_CHUNK,), jnp.float32),
            pltpu.SemaphoreType.DMA,
            pltpu.SemaphoreType.DMA,
            pltpu.SemaphoreType.DMA,
        ],
    )
    def sc_kernel(center_hbm, ctx_hbm, cen_tab, ctx_tab, out_hbm,
                  idx_c, v_rows, idx_v0, idx_v1, u_rows0, u_rows1,
                  tbuf, res0, res1, gsem0, gsem1, osem):
        wid = lax.axis_index("s") * _NC + lax.axis_index("c")
        iota = lax.iota(jnp.int32, 16)
        bufs = ((idx_v0, u_rows0, res0, gsem0),
                (idx_v1, u_rows1, res1, gsem1))

        pltpu.sync_copy(center_hbm.at[pl.ds(wid * _BPW, _BPW)], idx_c)
        cps = [
            pltpu.async_copy(cen_tab.at[idx_c.at[pl.ds(k * 128, 128)]],
                             v_rows.at[pl.ds(k * 128, 128), :], gsem0)
            for k in range(_BPW // 128)
        ]
        for cp in cps:
            cp.wait()

        base_w = wid * (_BPW * _L)

        def fetch(g, buf):
            idx, u, _, sem = bufs[buf]
            base = base_w + g * _CHUNK
            pltpu.sync_copy(ctx_hbm.at[pl.ds(base, _CHUNK)], idx)
            return [
                pltpu.async_copy(
                    ctx_tab.at[idx.at[pl.ds(j * _GSUB, _GSUB)]],
                    u.at[pl.ds(j * _GSUB, _GSUB), :],
                    sem)
                for j in range(_CHUNK // _GSUB)
            ]

        def drain(buf):
            idx, u, _, sem = bufs[buf]
            zc = pltpu.make_async_copy(
                ctx_tab.at[idx.at[pl.ds(0, _GSUB)]],
                u.at[pl.ds(0, _GSUB), :], sem)
            for _ in range(_CHUNK // _GSUB):
                zc.wait()

        def compute(g, buf):
            _, u, r, _ = bufs[buf]
            gb0 = g * _CB
            for b in range(_CB):
                gb = gb0 + b
                vv = [v_rows[gb, pl.ds(dc * 16, 16)] for dc in range(4)]
                for l0, nl in ((0, 16), (16, 16), (32, 16), (48, 2)):
                    for li in range(nl):
                        row = b * _L + l0 + li
                        p = u[row, pl.ds(0, 16)] * vv[0]
                        for dc in range(1, 4):
                            p = p + u[row, pl.ds(dc * 16, 16)] * vv[dc]
                        plsc.store_scatter(tbuf, [iota + li * _TP], p)
                    acc = plsc.load_gather(tbuf, [iota * _TP])
                    for jj in range(1, 16):
                        acc = acc + plsc.load_gather(tbuf, [iota * _TP + jj])
                    oidx = iota + (b * _L + l0)
                    if nl == 16:
                        plsc.store_scatter(r, [oidx], acc)
                    else:
                        m = iota < nl
                        plsc.store_scatter(r, [jnp.where(m, oidx, 0)], acc,
                                           mask=m)

        fetch(0, 0)

        def pair_body(p, carry):
            g0 = p * 2
            fetch(g0 + 1, 1)
            drain(0)
            compute(g0, 0)
            pltpu.async_copy(res0,
                             out_hbm.at[pl.ds(base_w + g0 * _CHUNK, _CHUNK)],
                             osem)
            @pl.when(p < _NCHUNK // 2 - 1)
            def _():
                fetch(g0 + 2, 0)
            drain(1)
            compute(g0 + 1, 1)
            pltpu.async_copy(res1,
                             out_hbm.at[pl.ds(base_w + (g0 + 1) * _CHUNK,
                                              _CHUNK)],
                             osem)
            pltpu.make_async_copy(
                res0, out_hbm.at[pl.ds(base_w + g0 * _CHUNK, _CHUNK)],
                osem).wait()
            pltpu.make_async_copy(
                res1,
                out_hbm.at[pl.ds(base_w + (g0 + 1) * _CHUNK, _CHUNK)],
                osem).wait()
            return carry

        lax.fori_loop(0, _NCHUNK // 2, pair_body, 0)

    return sc_kernel


@functools.cache
def _sc_kernel_cached():
    return _make_sc_kernel()


def kernel(center, contexts_and_negatives, central_table, context_table):
    cen_p = jnp.pad(central_table, ((0, _VPAD - _V - 1), (0, _W - _D)))
    ctx_p = jnp.pad(context_table, ((0, _VPAD - _V - 1), (0, _W - _D)))
    out_flat = _sc_kernel_cached()(center.reshape(-1),
                                   contexts_and_negatives.reshape(-1),
                                   cen_p, ctx_p)
    return out_flat.reshape(_B, 1, _L)


# speedup vs baseline: 1.1734x; 1.0304x over previous
import functools

import jax
import jax.numpy as jnp
from jax import lax
from jax.experimental import pallas as pl
from jax.experimental.pallas import tpu as pltpu
from jax.experimental.pallas import tpu_sc as plsc

_B = 16384
_L = 50
_D = 64
_V = 1000000
_VPAD = 1000008
_W = 128

_NC = 2
_NS = 16
_NW = _NC * _NS
_BPW = _B // _NW
_NPH = 4
_BPP = _BPW // _NPH
_CB = 4
_CHUNK = _CB * _L
_NCH = _BPP // _CB
_SLAB = _BPP * _L
_SLABPAD = _SLAB + 16
_GSUB = 40
_TP = 17


def _make_sc_kernel():
    mesh = plsc.VectorSubcoreMesh(core_axis_name="c", subcore_axis_name="s")

    @functools.partial(
        pl.kernel,
        mesh=mesh,
        compiler_params=pltpu.CompilerParams(needs_layout_passes=False,
                                             use_tc_tiling_on_sc=True),
        out_type=jax.ShapeDtypeStruct((_B * _L,), jnp.float32),
        scratch_types=[
            pltpu.VMEM((_BPP,), jnp.int32),
            pltpu.VMEM((_BPP, _W), jnp.float32),
            pltpu.VMEM((_SLABPAD,), jnp.int32),
            pltpu.VMEM((_CHUNK + 8,), jnp.int32),
            pltpu.VMEM((_CHUNK + 8,), jnp.int32),
            pltpu.VMEM((_CHUNK, _W), jnp.float32),
            pltpu.VMEM((_CHUNK, _W), jnp.float32),
            pltpu.VMEM((16 * _TP,), jnp.float32),
            pltpu.VMEM((_SLAB,), jnp.float32),
            pltpu.SemaphoreType.DMA,
            pltpu.SemaphoreType.DMA,
            pltpu.SemaphoreType.DMA,
        ],
    )
    def sc_kernel(center_hbm, ctx_hbm, cen_tab, ctx_tab, out_hbm,
                  idx_c, v_rows, idx_slab, idx_ch0, idx_ch1,
                  u_rows0, u_rows1, tbuf, res, gsem0, gsem1, vsem):
        wid = lax.axis_index("s") * _NC + lax.axis_index("c")
        iota = lax.iota(jnp.int32, 16)
        bufs = ((idx_ch0, u_rows0, gsem0), (idx_ch1, u_rows1, gsem1))

        def stage_and_fetch(c, buf):
            idx, u, sem = bufs[buf]
            for k in range(_CHUNK // 16 + 1):
                idx[pl.ds(k * 16, 16)] = idx_slab[pl.ds(c * _CHUNK + k * 16,
                                                        16)]
            for j in range(_CHUNK // _GSUB):
                pltpu.async_copy(ctx_tab.at[idx.at[pl.ds(j * _GSUB, _GSUB)]],
                                 u.at[pl.ds(j * _GSUB, _GSUB), :], sem)

        def drain(buf):
            idx, u, sem = bufs[buf]
            zc = pltpu.make_async_copy(
                ctx_tab.at[idx.at[pl.ds(0, _GSUB)]],
                u.at[pl.ds(0, _GSUB), :], sem)
            for _ in range(_CHUNK // _GSUB):
                zc.wait()

        def compute(c, buf):
            _, u, _ = bufs[buf]
            for b in range(_CB):
                gb = c * _CB + b
                vv = [v_rows[gb, pl.ds(dc * 16, 16)] for dc in range(4)]
                for l0, nl in ((0, 16), (16, 16), (32, 16), (48, 2)):
                    for li in range(nl):
                        row = b * _L + l0 + li
                        p = u[row, pl.ds(0, 16)] * vv[0]
                        for dc in range(1, 4):
                            p = p + u[row, pl.ds(dc * 16, 16)] * vv[dc]
                        plsc.store_scatter(tbuf, [iota + li * _TP], p)
                    acc = plsc.load_gather(tbuf, [iota * _TP])
                    for jj in range(1, 16):
                        acc = acc + plsc.load_gather(tbuf, [iota * _TP + jj])
                    oidx = iota + (c * _CHUNK + b * _L + l0)
                    if nl == 16:
                        plsc.store_scatter(res, [oidx], acc)
                    else:
                        m = iota < nl
                        plsc.store_scatter(res, [jnp.where(m, oidx, 0)], acc,
                                           mask=m)

        def phase_body(ph, carry):
            pbase = wid * (_BPW * _L) + ph * _SLAB
            pltpu.sync_copy(ctx_hbm.at[pl.ds(pbase, _SLAB)],
                            idx_slab.at[pl.ds(0, _SLAB)])
            pltpu.sync_copy(center_hbm.at[pl.ds(wid * _BPW + ph * _BPP,
                                                _BPP)], idx_c)
            vcp = pltpu.async_copy(cen_tab.at[idx_c], v_rows, vsem)
            stage_and_fetch(0, 0)
            vcp.wait()

            def pair_body(q, carry2):
                c0 = q * 2
                stage_and_fetch(c0 + 1, 1)
                drain(0)
                compute(c0, 0)

                @pl.when(q < _NCH // 2 - 1)
                def _():
                    stage_and_fetch(c0 + 2, 0)
                drain(1)
                compute(c0 + 1, 1)
                return carry2

            lax.fori_loop(0, _NCH // 2, pair_body, 0)
            pltpu.sync_copy(res, out_hbm.at[pl.ds(pbase, _SLAB)])
            return carry

        lax.fori_loop(0, _NPH, phase_body, 0)

    return sc_kernel


@functools.cache
def _sc_kernel_cached():
    return _make_sc_kernel()


def kernel(center, contexts_and_negatives, central_table, context_table):
    cen_p = jnp.pad(central_table, ((0, _VPAD - _V - 1), (0, _W - _D)))
    ctx_p = jnp.pad(context_table, ((0, _VPAD - _V - 1), (0, _W - _D)))
    out_flat = _sc_kernel_cached()(center.reshape(-1),
                                   contexts_and_negatives.reshape(-1),
                                   cen_p, ctx_p)
    return out_flat.reshape(_B, 1, _L)


# speedup vs baseline: 1.2955x; 1.1041x over previous
import functools

import jax
import jax.numpy as jnp
from jax import lax
from jax.experimental import pallas as pl
from jax.experimental.pallas import tpu as pltpu
from jax.experimental.pallas import tpu_sc as plsc

_B = 16384
_L = 50
_D = 64
_V = 1000000
_W = 128

_NC = 2
_NS = 16
_NW = _NC * _NS
_BPW = _B // _NW
_NPH = 4
_BPP = _BPW // _NPH
_CB = 4
_CHUNK = _CB * _L
_NCH = _BPP // _CB
_SLAB = _BPP * _L
_SLABPAD = _SLAB + 16
_GSUB = 40
_TP = 17


def _make_sc_kernel():
    mesh = plsc.VectorSubcoreMesh(core_axis_name="c", subcore_axis_name="s")

    @functools.partial(
        pl.kernel,
        mesh=mesh,
        compiler_params=pltpu.CompilerParams(needs_layout_passes=False,
                                             use_tc_tiling_on_sc=True),
        out_type=jax.ShapeDtypeStruct((_B * _L,), jnp.float32),
        scratch_types=[
            pltpu.VMEM((_BPP,), jnp.int32),
            pltpu.VMEM((_BPP, _W), jnp.float32),
            pltpu.VMEM((_SLABPAD,), jnp.int32),
            pltpu.VMEM((_CHUNK + 8,), jnp.int32),
            pltpu.VMEM((_CHUNK + 8,), jnp.int32),
            pltpu.VMEM((_CHUNK, _W), jnp.float32),
            pltpu.VMEM((_CHUNK, _W), jnp.float32),
            pltpu.VMEM((16 * _TP,), jnp.float32),
            pltpu.VMEM((_SLAB,), jnp.float32),
            pltpu.SemaphoreType.DMA,
            pltpu.SemaphoreType.DMA,
            pltpu.SemaphoreType.DMA,
        ],
    )
    def sc_kernel(center_hbm, ctx_hbm, tab, out_hbm,
                  idx_c, v_rows, idx_slab, idx_ch0, idx_ch1,
                  u_rows0, u_rows1, tbuf, res, gsem0, gsem1, vsem):
        wid = lax.axis_index("s") * _NC + lax.axis_index("c")
        iota = lax.iota(jnp.int32, 16)
        bufs = ((idx_ch0, u_rows0, gsem0), (idx_ch1, u_rows1, gsem1))

        def stage_and_fetch(c, buf):
            idx, u, sem = bufs[buf]
            for k in range(_CHUNK // 16 + 1):
                idx[pl.ds(k * 16, 16)] = idx_slab[pl.ds(c * _CHUNK + k * 16,
                                                        16)]
            for j in range(_CHUNK // _GSUB):
                pltpu.async_copy(tab.at[idx.at[pl.ds(j * _GSUB, _GSUB)]],
                                 u.at[pl.ds(j * _GSUB, _GSUB), :], sem)

        def drain(buf):
            idx, u, sem = bufs[buf]
            zc = pltpu.make_async_copy(
                tab.at[idx.at[pl.ds(0, _GSUB)]],
                u.at[pl.ds(0, _GSUB), :], sem)
            for _ in range(_CHUNK // _GSUB):
                zc.wait()

        def compute(c, buf):
            _, u, _ = bufs[buf]
            for b in range(_CB):
                gb = c * _CB + b
                vv = [v_rows[gb, pl.ds(dc * 16, 16)] for dc in range(4)]
                for l0, nl in ((0, 16), (16, 16), (32, 16), (48, 2)):
                    for li in range(nl):
                        row = b * _L + l0 + li
                        prods = [u[row, pl.ds(_D + dc * 16, 16)] * vv[dc]
                                 for dc in range(4)]
                        p = (prods[0] + prods[1]) + (prods[2] + prods[3])
                        plsc.store_scatter(tbuf, [iota + li * _TP], p)
                    cols = [plsc.load_gather(tbuf, [iota * _TP + jj])
                            for jj in range(16)]
                    while len(cols) > 1:
                        cols = [cols[i] + cols[i + 1]
                                for i in range(0, len(cols), 2)]
                    acc = cols[0]
                    oidx = iota + (c * _CHUNK + b * _L + l0)
                    if nl == 16:
                        plsc.store_scatter(res, [oidx], acc)
                    else:
                        m = iota < nl
                        plsc.store_scatter(res, [jnp.where(m, oidx, 0)], acc,
                                           mask=m)

        def phase_body(ph, carry):
            pbase = wid * (_BPW * _L) + ph * _SLAB
            pltpu.sync_copy(ctx_hbm.at[pl.ds(pbase, _SLAB)],
                            idx_slab.at[pl.ds(0, _SLAB)])
            pltpu.sync_copy(center_hbm.at[pl.ds(wid * _BPW + ph * _BPP,
                                                _BPP)], idx_c)
            vcp = pltpu.async_copy(tab.at[idx_c], v_rows, vsem)
            stage_and_fetch(0, 0)
            vcp.wait()

            def pair_body(q, carry2):
                c0 = q * 2
                stage_and_fetch(c0 + 1, 1)
                drain(0)
                compute(c0, 0)

                @pl.when(q < _NCH // 2 - 1)
                def _():
                    stage_and_fetch(c0 + 2, 0)
                drain(1)
                compute(c0 + 1, 1)
                return carry2

            lax.fori_loop(0, _NCH // 2, pair_body, 0)
            pltpu.sync_copy(res, out_hbm.at[pl.ds(pbase, _SLAB)])
            return carry

        lax.fori_loop(0, _NPH, phase_body, 0)

    return sc_kernel


@functools.cache
def _sc_kernel_cached():
    return _make_sc_kernel()


def kernel(center, contexts_and_negatives, central_table, context_table):
    tab = jnp.concatenate([central_table[:_V], context_table[:_V]], axis=1)
    out_flat = _sc_kernel_cached()(center.reshape(-1),
                                   contexts_and_negatives.reshape(-1),
                                   tab)
    return out_flat.reshape(_B, 1, _L)


# speedup vs baseline: 1.2982x; 1.0021x over previous
import functools

import jax
import jax.numpy as jnp
from jax import lax
from jax.experimental import pallas as pl
from jax.experimental.pallas import tpu as pltpu
from jax.experimental.pallas import tpu_sc as plsc

_B = 16384
_L = 50
_D = 64
_V = 1000000
_W = 128

_NC = 2
_NS = 16
_NW = _NC * _NS
_BPW = _B // _NW
_NPH = 4
_BPP = _BPW // _NPH
_CB = 2
_CHUNK = _CB * _L
_NCH = _BPP // _CB
_NBUF = 4
_SLAB = _BPP * _L
_SLABPAD = _SLAB + 16
_IDXPAD = 112
_TP = 17


def _make_sc_kernel():
    mesh = plsc.VectorSubcoreMesh(core_axis_name="c", subcore_axis_name="s")

    @functools.partial(
        pl.kernel,
        mesh=mesh,
        compiler_params=pltpu.CompilerParams(needs_layout_passes=False,
                                             use_tc_tiling_on_sc=True),
        out_type=jax.ShapeDtypeStruct((_B * _L,), jnp.float32),
        scratch_types=[
            pltpu.VMEM((_BPP,), jnp.int32),
            pltpu.VMEM((_BPP, _W), jnp.float32),
            pltpu.VMEM((_SLABPAD,), jnp.int32),
            pltpu.VMEM((_IDXPAD,), jnp.int32),
            pltpu.VMEM((_IDXPAD,), jnp.int32),
            pltpu.VMEM((_IDXPAD,), jnp.int32),
            pltpu.VMEM((_IDXPAD,), jnp.int32),
            pltpu.VMEM((_CHUNK, _W), jnp.float32),
            pltpu.VMEM((_CHUNK, _W), jnp.float32),
            pltpu.VMEM((_CHUNK, _W), jnp.float32),
            pltpu.VMEM((_CHUNK, _W), jnp.float32),
            pltpu.VMEM((16 * _TP,), jnp.float32),
            pltpu.VMEM((_SLAB,), jnp.float32),
            pltpu.SemaphoreType.DMA,
            pltpu.SemaphoreType.DMA,
            pltpu.SemaphoreType.DMA,
            pltpu.SemaphoreType.DMA,
            pltpu.SemaphoreType.DMA,
        ],
    )
    def sc_kernel(center_hbm, ctx_hbm, tab, out_hbm,
                  idx_c, v_rows, idx_slab, idx_ch0, idx_ch1, idx_ch2, idx_ch3,
                  u_rows0, u_rows1, u_rows2, u_rows3, tbuf, res,
                  gsem0, gsem1, gsem2, gsem3, vsem):
        wid = lax.axis_index("s") * _NC + lax.axis_index("c")
        iota = lax.iota(jnp.int32, 16)
        bufs = ((idx_ch0, u_rows0, gsem0), (idx_ch1, u_rows1, gsem1),
                (idx_ch2, u_rows2, gsem2), (idx_ch3, u_rows3, gsem3))

        def stage_and_fetch(c, buf):
            idx, u, sem = bufs[buf]
            for k in range(_IDXPAD // 16):
                idx[pl.ds(k * 16, 16)] = idx_slab[pl.ds(c * _CHUNK + k * 16,
                                                        16)]
            pltpu.async_copy(tab.at[idx.at[pl.ds(0, _CHUNK)]], u, sem)

        def drain(buf):
            idx, u, sem = bufs[buf]
            pltpu.make_async_copy(
                tab.at[idx.at[pl.ds(0, _CHUNK)]], u, sem).wait()

        def compute(c, buf):
            _, u, _ = bufs[buf]
            for b in range(_CB):
                gb = c * _CB + b
                vv = [v_rows[gb, pl.ds(dc * 16, 16)] for dc in range(4)]
                for l0, nl in ((0, 16), (16, 16), (32, 16), (48, 2)):
                    for li in range(nl):
                        row = b * _L + l0 + li
                        prods = [u[row, pl.ds(_D + dc * 16, 16)] * vv[dc]
                                 for dc in range(4)]
                        p = (prods[0] + prods[1]) + (prods[2] + prods[3])
                        plsc.store_scatter(tbuf, [iota + li * _TP], p)
                    cols = [plsc.load_gather(tbuf, [iota * _TP + jj])
                            for jj in range(16)]
                    while len(cols) > 1:
                        cols = [cols[i] + cols[i + 1]
                                for i in range(0, len(cols), 2)]
                    acc = cols[0]
                    oidx = iota + (c * _CHUNK + b * _L + l0)
                    if nl == 16:
                        plsc.store_scatter(res, [oidx], acc)
                    else:
                        m = iota < nl
                        plsc.store_scatter(res, [jnp.where(m, oidx, 0)], acc,
                                           mask=m)

        def phase_body(ph, carry):
            pbase = wid * (_BPW * _L) + ph * _SLAB
            pltpu.sync_copy(ctx_hbm.at[pl.ds(pbase, _SLAB)],
                            idx_slab.at[pl.ds(0, _SLAB)])
            pltpu.sync_copy(center_hbm.at[pl.ds(wid * _BPW + ph * _BPP,
                                                _BPP)], idx_c)
            vcp = pltpu.async_copy(tab.at[idx_c], v_rows, vsem)
            for k in range(_NBUF):
                stage_and_fetch(k, k)
            vcp.wait()

            def ring_body(q, carry2):
                c0 = q * _NBUF
                for k in range(_NBUF):
                    c = c0 + k
                    drain(k)
                    compute(c, k)

                    @pl.when(c + _NBUF < _NCH)
                    def _():
                        stage_and_fetch(c + _NBUF, k)
                return carry2

            lax.fori_loop(0, _NCH // _NBUF, ring_body, 0)
            pltpu.sync_copy(res, out_hbm.at[pl.ds(pbase, _SLAB)])
            return carry

        lax.fori_loop(0, _NPH, phase_body, 0)

    return sc_kernel


@functools.cache
def _sc_kernel_cached():
    return _make_sc_kernel()


def kernel(center, contexts_and_negatives, central_table, context_table):
    tab = jnp.concatenate([central_table[:_V], context_table[:_V]], axis=1)
    out_flat = _sc_kernel_cached()(center.reshape(-1),
                                   contexts_and_negatives.reshape(-1),
                                   tab)
    return out_flat.reshape(_B, 1, _L)


# speedup vs baseline: 1.3136x; 1.0118x over previous
import functools

import jax
import jax.numpy as jnp
from jax import lax
from jax.experimental import pallas as pl
from jax.experimental.pallas import tpu as pltpu
from jax.experimental.pallas import tpu_sc as plsc

_B = 16384
_L = 50
_D = 64
_V = 1000000
_W = 128

_NC = 2
_NS = 16
_NW = _NC * _NS
_BPW = _B // _NW
_NPH = 2
_BPP = _BPW // _NPH
_CB = 2
_CHUNK = _CB * _L
_NCH = _BPP // _CB
_NBUF = 4
_SLAB = _BPP * _L
_SLABPAD = _SLAB + 16
_IDXPAD = 112
_TP = 17


def _make_sc_kernel():
    mesh = plsc.VectorSubcoreMesh(core_axis_name="c", subcore_axis_name="s")

    @functools.partial(
        pl.kernel,
        mesh=mesh,
        compiler_params=pltpu.CompilerParams(needs_layout_passes=False,
                                             use_tc_tiling_on_sc=True),
        out_type=jax.ShapeDtypeStruct((_B * _L,), jnp.float32),
        scratch_types=[
            pltpu.VMEM((_BPP,), jnp.int32),
            pltpu.VMEM((_BPP, _W), jnp.float32),
            pltpu.VMEM((_SLABPAD,), jnp.int32),
            pltpu.VMEM((_IDXPAD,), jnp.int32),
            pltpu.VMEM((_IDXPAD,), jnp.int32),
            pltpu.VMEM((_IDXPAD,), jnp.int32),
            pltpu.VMEM((_IDXPAD,), jnp.int32),
            pltpu.VMEM((_CHUNK, _W), jnp.float32),
            pltpu.VMEM((_CHUNK, _W), jnp.float32),
            pltpu.VMEM((_CHUNK, _W), jnp.float32),
            pltpu.VMEM((_CHUNK, _W), jnp.float32),
            pltpu.VMEM((16 * _TP,), jnp.float32),
            pltpu.VMEM((_SLAB,), jnp.float32),
            pltpu.SemaphoreType.DMA,
            pltpu.SemaphoreType.DMA,
            pltpu.SemaphoreType.DMA,
            pltpu.SemaphoreType.DMA,
            pltpu.SemaphoreType.DMA,
        ],
    )
    def sc_kernel(center_hbm, ctx_hbm, tab, out_hbm,
                  idx_c, v_rows, idx_slab, idx_ch0, idx_ch1, idx_ch2, idx_ch3,
                  u_rows0, u_rows1, u_rows2, u_rows3, tbuf, res,
                  gsem0, gsem1, gsem2, gsem3, vsem):
        wid = lax.axis_index("s") * _NC + lax.axis_index("c")
        iota = lax.iota(jnp.int32, 16)
        bufs = ((idx_ch0, u_rows0, gsem0), (idx_ch1, u_rows1, gsem1),
                (idx_ch2, u_rows2, gsem2), (idx_ch3, u_rows3, gsem3))

        def stage_and_fetch(c, buf):
            idx, u, sem = bufs[buf]
            for k in range(_IDXPAD // 16):
                idx[pl.ds(k * 16, 16)] = idx_slab[pl.ds(c * _CHUNK + k * 16,
                                                        16)]
            pltpu.async_copy(tab.at[idx.at[pl.ds(0, _CHUNK)]], u, sem)

        def drain(buf):
            idx, u, sem = bufs[buf]
            pltpu.make_async_copy(
                tab.at[idx.at[pl.ds(0, _CHUNK)]], u, sem).wait()

        def compute(c, buf):
            _, u, _ = bufs[buf]
            for b in range(_CB):
                gb = c * _CB + b
                vv = [v_rows[gb, pl.ds(dc * 16, 16)] for dc in range(4)]
                for l0, nl in ((0, 16), (16, 16), (32, 16), (48, 2)):
                    for li in range(nl):
                        row = b * _L + l0 + li
                        prods = [u[row, pl.ds(_D + dc * 16, 16)] * vv[dc]
                                 for dc in range(4)]
                        p = (prods[0] + prods[1]) + (prods[2] + prods[3])
                        plsc.store_scatter(tbuf, [iota + li * _TP], p)
                    cols = [plsc.load_gather(tbuf, [iota * _TP + jj])
                            for jj in range(16)]
                    while len(cols) > 1:
                        cols = [cols[i] + cols[i + 1]
                                for i in range(0, len(cols), 2)]
                    acc = cols[0]
                    oidx = iota + (c * _CHUNK + b * _L + l0)
                    if nl == 16:
                        plsc.store_scatter(res, [oidx], acc)
                    else:
                        m = iota < nl
                        plsc.store_scatter(res, [jnp.where(m, oidx, 0)], acc,
                                           mask=m)

        def phase_body(ph, carry):
            pbase = wid * (_BPW * _L) + ph * _SLAB
            pltpu.sync_copy(ctx_hbm.at[pl.ds(pbase, _SLAB)],
                            idx_slab.at[pl.ds(0, _SLAB)])
            pltpu.sync_copy(center_hbm.at[pl.ds(wid * _BPW + ph * _BPP,
                                                _BPP)], idx_c)
            vcp = pltpu.async_copy(tab.at[idx_c], v_rows, vsem)
            for k in range(_NBUF):
                stage_and_fetch(k, k)
            vcp.wait()

            def ring_body(q, carry2):
                c0 = q * _NBUF
                for k in range(_NBUF):
                    c = c0 + k
                    drain(k)
                    compute(c, k)

                    @pl.when(c + _NBUF < _NCH)
                    def _():
                        stage_and_fetch(c + _NBUF, k)
                return carry2

            lax.fori_loop(0, _NCH // _NBUF, ring_body, 0)
            pltpu.sync_copy(res, out_hbm.at[pl.ds(pbase, _SLAB)])
            return carry

        lax.fori_loop(0, _NPH, phase_body, 0)

    return sc_kernel


@functools.cache
def _sc_kernel_cached():
    return _make_sc_kernel()


def kernel(center, contexts_and_negatives, central_table, context_table):
    tab = jnp.concatenate([central_table[:_V], context_table[:_V]], axis=1)
    out_flat = _sc_kernel_cached()(center.reshape(-1),
                                   contexts_and_negatives.reshape(-1),
                                   tab)
    return out_flat.reshape(_B, 1, _L)


# speedup vs baseline: 1.3211x; 1.0057x over previous
import functools

import jax
import jax.numpy as jnp
from jax import lax
from jax.experimental import pallas as pl
from jax.experimental.pallas import tpu as pltpu
from jax.experimental.pallas import tpu_sc as plsc

_B = 16384
_L = 50
_D = 64
_V = 1000000
_W = 128

_NC = 2
_NS = 16
_NW = _NC * _NS
_BPW = _B // _NW
_NPH = 2
_BPP = _BPW // _NPH
_CB = 2
_CHUNK = _CB * _L
_NCH = _BPP // _CB
_NBUF = 4
_SLAB = _BPP * _L
_SLABPAD = _SLAB + 16
_IDXPAD = 112
_TP = 17


def _make_sc_kernel():
    mesh = plsc.VectorSubcoreMesh(core_axis_name="c", subcore_axis_name="s")

    @functools.partial(
        pl.kernel,
        mesh=mesh,
        compiler_params=pltpu.CompilerParams(needs_layout_passes=False,
                                             use_tc_tiling_on_sc=True),
        out_type=jax.ShapeDtypeStruct((_B * _L,), jnp.float32),
        scratch_types=[
            pltpu.VMEM((_BPP,), jnp.int32),
            pltpu.VMEM((_BPP, _W), jnp.float32),
            pltpu.VMEM((_SLABPAD,), jnp.int32),
            pltpu.VMEM((_IDXPAD,), jnp.int32),
            pltpu.VMEM((_IDXPAD,), jnp.int32),
            pltpu.VMEM((_IDXPAD,), jnp.int32),
            pltpu.VMEM((_IDXPAD,), jnp.int32),
            pltpu.VMEM((_CHUNK, _W), jnp.float32),
            pltpu.VMEM((_CHUNK, _W), jnp.float32),
            pltpu.VMEM((_CHUNK, _W), jnp.float32),
            pltpu.VMEM((_CHUNK, _W), jnp.float32),
            pltpu.VMEM((16 * _TP,), jnp.float32),
            pltpu.VMEM((_SLAB,), jnp.float32),
            pltpu.SemaphoreType.DMA,
            pltpu.SemaphoreType.DMA,
            pltpu.SemaphoreType.DMA,
            pltpu.SemaphoreType.DMA,
            pltpu.SemaphoreType.DMA,
        ],
    )
    def sc_kernel(center_hbm, ctx_hbm, tab, out_hbm,
                  idx_c, v_rows, idx_slab, idx_ch0, idx_ch1, idx_ch2, idx_ch3,
                  u_rows0, u_rows1, u_rows2, u_rows3, tbuf, res,
                  gsem0, gsem1, gsem2, gsem3, vsem):
        wid = lax.axis_index("s") * _NC + lax.axis_index("c")
        iota = lax.iota(jnp.int32, 16)
        bufs = ((idx_ch0, u_rows0, gsem0), (idx_ch1, u_rows1, gsem1),
                (idx_ch2, u_rows2, gsem2), (idx_ch3, u_rows3, gsem3))

        def stage_and_fetch(c, buf):
            idx, u, sem = bufs[buf]
            for k in range(_IDXPAD // 16):
                idx[pl.ds(k * 16, 16)] = idx_slab[pl.ds(c * _CHUNK + k * 16,
                                                        16)]
            pltpu.async_copy(tab.at[idx.at[pl.ds(0, _CHUNK)]], u, sem)

        def drain(buf):
            idx, u, sem = bufs[buf]
            pltpu.make_async_copy(
                tab.at[idx.at[pl.ds(0, _CHUNK)]], u, sem).wait()

        def compute(c, buf):
            _, u, _ = bufs[buf]
            for b in range(_CB):
                gb = c * _CB + b
                vv = [v_rows[gb, pl.ds(dc * 16, 16)] for dc in range(4)]
                for l0, nl in ((0, 16), (16, 16), (32, 16), (48, 2)):
                    for li in range(nl):
                        row = b * _L + l0 + li
                        prods = [u[row, pl.ds(_D + dc * 16, 16)] * vv[dc]
                                 for dc in range(4)]
                        p = (prods[0] + prods[1]) + (prods[2] + prods[3])
                        plsc.store_scatter(tbuf, [iota + li * _TP], p)
                    cols = [plsc.load_gather(tbuf, [iota * _TP + jj])
                            for jj in range(16)]
                    while len(cols) > 1:
                        cols = [cols[i] + cols[i + 1]
                                for i in range(0, len(cols), 2)]
                    acc = cols[0]
                    oidx = iota + (c * _CHUNK + b * _L + l0)
                    if nl == 16:
                        plsc.store_scatter(res, [oidx], acc)
                    else:
                        m = iota < nl
                        plsc.store_scatter(res, [jnp.where(m, oidx, 0)], acc,
                                           mask=m)

        def phase_body(ph, carry):
            pbase = wid * (_BPW * _L) + ph * _SLAB
            pltpu.sync_copy(ctx_hbm.at[pl.ds(pbase, _SLAB)],
                            idx_slab.at[pl.ds(0, _SLAB)])
            pltpu.sync_copy(center_hbm.at[pl.ds(wid * _BPW + ph * _BPP,
                                                _BPP)], idx_c)
            vcps = [
                pltpu.async_copy(tab.at[idx_c.at[pl.ds(k * 128, 128)]],
                                 v_rows.at[pl.ds(k * 128, 128), :], vsem)
                for k in range(_BPP // 128)
            ]
            for k in range(_NBUF):
                stage_and_fetch(k, k)
            for vcp in vcps:
                vcp.wait()

            def ring_body(q, carry2):
                c0 = q * _NBUF
                for k in range(_NBUF):
                    c = c0 + k
                    drain(k)
                    compute(c, k)

                    @pl.when(c + _NBUF < _NCH)
                    def _():
                        stage_and_fetch(c + _NBUF, k)
                return carry2

            lax.fori_loop(0, _NCH // _NBUF, ring_body, 0)
            pltpu.sync_copy(res, out_hbm.at[pl.ds(pbase, _SLAB)])
            return carry

        lax.fori_loop(0, _NPH, phase_body, 0)

    return sc_kernel


@functools.cache
def _sc_kernel_cached():
    return _make_sc_kernel()


def kernel(center, contexts_and_negatives, central_table, context_table):
    tab = jnp.concatenate([central_table[:_V], context_table[:_V]], axis=1)
    out_flat = _sc_kernel_cached()(center.reshape(-1),
                                   contexts_and_negatives.reshape(-1),
                                   tab)
    return out_flat.reshape(_B, 1, _L)
